# Initial kernel scaffold; baseline (speedup 1.0000x reference)
import jax
import jax.numpy as jnp
from jax.experimental import pallas as pl


def kernel(memory, last_update_t, edge_ts, edge_feats, time_w, time_b, W_ih, W_hh, b_ih, b_hh, nid, edge_src, edge_dst):
    raise NotImplementedError("write your pallas kernel here")


# speedup vs baseline: 23.5703x; 23.5703x over previous
import functools
import jax
import jax.numpy as jnp
from jax import lax
from jax.experimental import pallas as pl
from jax.experimental.pallas import tpu as pltpu
from jax.experimental.pallas import tpu_sc as plsc

_N_NODES = 10000
_N_EDGES = 320000
_MEM_DIM = 128
_E_FEAT = 16
_T_DIM = 100

_NC = 2
_NS = 16
_NW = _NC * _NS
_N_PAD = 10240
_BINS_W = _N_PAD // _NW
_EDGES_W = _N_EDGES // _NW
_L = 16

_f32 = jnp.float32
_i32 = jnp.int32


def _seg_argmax_partial(ts_hbm, dst_hbm, pts_hbm, pid_hbm,
                        ts_v, dst_v, bt_v, be_v):
    wid = lax.axis_index("s") * _NC + lax.axis_index("c")
    ebase = wid * _EDGES_W
    pltpu.sync_copy(ts_hbm.at[pl.ds(ebase, _EDGES_W)], ts_v)
    pltpu.sync_copy(dst_hbm.at[pl.ds(ebase, _EDGES_W)], dst_v)

    neg = jnp.full((_L,), -jnp.inf, _f32)
    mone = jnp.full((_L,), -1, _i32)

    def init_body(i, c):
        bt_v[pl.ds(i * _L, _L)] = neg
        be_v[pl.ds(i * _L, _L)] = mone
        return c
    lax.fori_loop(0, _N_PAD // _L, init_body, 0)

    iota = lax.iota(_i32, _L)

    def edge_body(i, c):
        off = i * _L
        d = dst_v[pl.ds(off, _L)]
        t = ts_v[pl.ds(off, _L)]
        e = iota + (ebase + off)

        def w_body(r, cc):
            at = plsc.load_gather(bt_v, [d])
            ae = plsc.load_gather(be_v, [d])
            win = (t > at) | ((t == at) & (e > ae))
            plsc.store_scatter(bt_v, [d], t, mask=win)
            plsc.store_scatter(be_v, [d], e, mask=win)
            return cc

        lax.fori_loop(0, _L, w_body, 0)
        return c
    lax.fori_loop(0, _EDGES_W // _L, edge_body, 0)

    pltpu.sync_copy(bt_v, pts_hbm.at[wid])
    pltpu.sync_copy(be_v, pid_hbm.at[wid])


def _merge_and_gather(ptsf_hbm, pidf_hbm, esrc_hbm, efeatf_hbm, nid_hbm,
                      lut_hbm, mem_hbm,
                      best_o, segts_o, delta_o, nodets_o, featw_o,
                      ssrc_o, sdst_o,
                      bt_v, be_v, mt_v, me_v, bs_v, srcw_v, nsrc_v,
                      tsrc_v, nv_v, ntv_v, delta_v, fidx_v, featfl_v,
                      ssrc_v, sdst_v, sem):
    wid = lax.axis_index("s") * _NC + lax.axis_index("c")
    base = wid * _BINS_W
    nv = _BINS_W // _L

    pltpu.sync_copy(ptsf_hbm.at[pl.ds(base, _BINS_W)], bt_v)
    pltpu.sync_copy(pidf_hbm.at[pl.ds(base, _BINS_W)], be_v)

    def merge_body(t, c):
        pltpu.sync_copy(ptsf_hbm.at[pl.ds(t * _N_PAD + base, _BINS_W)], mt_v)
        pltpu.sync_copy(pidf_hbm.at[pl.ds(t * _N_PAD + base, _BINS_W)], me_v)

        def vb(v, cc):
            o = v * _L
            bt = bt_v[pl.ds(o, _L)]
            be = be_v[pl.ds(o, _L)]
            mt = mt_v[pl.ds(o, _L)]
            me = me_v[pl.ds(o, _L)]
            w = (mt > bt) | ((mt == bt) & (me > be))
            bt_v[pl.ds(o, _L)] = jnp.where(w, mt, bt)
            be_v[pl.ds(o, _L)] = jnp.where(w, me, be)
            return cc
        lax.fori_loop(0, nv, vb, 0)
        return c
    lax.fori_loop(1, _NW, merge_body, 0)

    def bs_body(v, c):
        o = v * _L
        bs_v[pl.ds(o, _L)] = jnp.maximum(be_v[pl.ds(o, _L)], 0)
        return c
    lax.fori_loop(0, nv, bs_body, 0)

    iota = lax.iota(_i32, _L)

    def fx_body(m, c):
        bs_vec = bs_v[pl.ds(m * _L, _L)]
        pos = iota * _E_FEAT + m * (_L * _E_FEAT)
        val = bs_vec * _E_FEAT
        for j in range(_E_FEAT):
            plsc.store_scatter(fidx_v, [pos + j], val + j)
        return c
    lax.fori_loop(0, nv, fx_body, 0)

    pltpu.async_copy(esrc_hbm.at[bs_v], srcw_v, sem).wait()
    pltpu.async_copy(efeatf_hbm.at[fidx_v], featfl_v, sem).wait()
    pltpu.async_copy(nid_hbm.at[srcw_v], nsrc_v, sem).wait()
    pltpu.async_copy(lut_hbm.at[nsrc_v], tsrc_v, sem).wait()
    pltpu.async_copy(mem_hbm.at[nsrc_v], ssrc_v, sem).wait()
    pltpu.sync_copy(nid_hbm.at[pl.ds(base, _BINS_W)], nv_v)
    pltpu.async_copy(lut_hbm.at[nv_v], ntv_v, sem).wait()
    pltpu.async_copy(mem_hbm.at[nv_v], sdst_v, sem).wait()

    def d_body(v, c):
        o = v * _L
        delta_v[pl.ds(o, _L)] = bt_v[pl.ds(o, _L)] - tsrc_v[pl.ds(o, _L)]
        return c
    lax.fori_loop(0, nv, d_body, 0)

    pltpu.sync_copy(be_v, best_o.at[pl.ds(base, _BINS_W)])
    pltpu.sync_copy(bt_v, segts_o.at[pl.ds(base, _BINS_W)])
    pltpu.sync_copy(delta_v, delta_o.at[pl.ds(base, _BINS_W)])
    pltpu.sync_copy(ntv_v, nodets_o.at[pl.ds(base, _BINS_W)])
    pltpu.sync_copy(featfl_v, featw_o.at[pl.ds(base * _E_FEAT, _BINS_W * _E_FEAT)])
    pltpu.sync_copy(ssrc_v, ssrc_o.at[pl.ds(base, _BINS_W)])
    pltpu.sync_copy(sdst_v, sdst_o.at[pl.ds(base, _BINS_W)])


def _gru_block(ssrc_ref, sdst_ref, featw_ref, delta_ref, segts_ref,
               nodets_ref, best_ref, tw_ref, tb_ref, wis_ref, wid_ref,
               wif_ref, wit_ref, whh_ref, bih_ref, bhh_ref,
               snew_ref, newts_ref):
    f32 = jnp.float32
    delta = delta_ref[...]
    te = jnp.cos(delta * tw_ref[...] + tb_ref[...])
    sdst = sdst_ref[...]
    gi = (jnp.dot(ssrc_ref[...], wis_ref[...], preferred_element_type=f32)
          + jnp.dot(sdst, wid_ref[...], preferred_element_type=f32)
          + jnp.dot(featw_ref[...], wif_ref[...], preferred_element_type=f32)
          + jnp.dot(te, wit_ref[...], preferred_element_type=f32)
          + bih_ref[...])
    gh = jnp.dot(sdst, whh_ref[...], preferred_element_type=f32) + bhh_ref[...]
    D = _MEM_DIM
    r = jax.nn.sigmoid(gi[:, :D] + gh[:, :D])
    z = jax.nn.sigmoid(gi[:, D:2 * D] + gh[:, D:2 * D])
    n = jnp.tanh(gi[:, 2 * D:] + r * gh[:, 2 * D:])
    s_all = (1.0 - z) * n + z * sdst
    has = best_ref[...] >= 0
    snew_ref[...] = jnp.where(has, s_all, sdst)
    newts_ref[...] = jnp.where(has, segts_ref[...], nodets_ref[...])


def kernel(memory, last_update_t, edge_ts, edge_feats, time_w, time_b,
           W_ih, W_hh, b_ih, b_hh, nid, edge_src, edge_dst):
    nid = nid.astype(_i32)
    edge_src = edge_src.astype(_i32)
    edge_dst = edge_dst.astype(_i32)
    nid_pad = jnp.pad(nid, (0, _N_PAD - _N_NODES))

    mesh = plsc.VectorSubcoreMesh(core_axis_name="c", subcore_axis_name="s")
    sc_params = pltpu.CompilerParams(needs_layout_passes=False)

    stage_a = functools.partial(
        pl.kernel,
        out_type=(jax.ShapeDtypeStruct((_NW, _N_PAD), _f32),
                  jax.ShapeDtypeStruct((_NW, _N_PAD), _i32)),
        mesh=mesh,
        scratch_types=[pltpu.VMEM((_EDGES_W,), _f32),
                       pltpu.VMEM((_EDGES_W,), _i32),
                       pltpu.VMEM((_N_PAD,), _f32),
                       pltpu.VMEM((_N_PAD,), _i32)],
        compiler_params=sc_params,
    )(_seg_argmax_partial)
    pts, pid = stage_a(edge_ts, edge_dst)

    stage_bc = functools.partial(
        pl.kernel,
        out_type=(jax.ShapeDtypeStruct((_N_PAD,), _i32),
                  jax.ShapeDtypeStruct((_N_PAD,), _f32),
                  jax.ShapeDtypeStruct((_N_PAD,), _f32),
                  jax.ShapeDtypeStruct((_N_PAD,), _f32),
                  jax.ShapeDtypeStruct((_N_PAD * _E_FEAT,), _f32),
                  jax.ShapeDtypeStruct((_N_PAD, _MEM_DIM), _f32),
                  jax.ShapeDtypeStruct((_N_PAD, _MEM_DIM), _f32)),
        mesh=mesh,
        scratch_types=[pltpu.VMEM((_BINS_W,), _f32),
                       pltpu.VMEM((_BINS_W,), _i32),
                       pltpu.VMEM((_BINS_W,), _f32),
                       pltpu.VMEM((_BINS_W,), _i32),
                       pltpu.VMEM((_BINS_W,), _i32),
                       pltpu.VMEM((_BINS_W,), _i32),
                       pltpu.VMEM((_BINS_W,), _i32),
                       pltpu.VMEM((_BINS_W,), _f32),
                       pltpu.VMEM((_BINS_W,), _i32),
                       pltpu.VMEM((_BINS_W,), _f32),
                       pltpu.VMEM((_BINS_W,), _f32),
                       pltpu.VMEM((_BINS_W * _E_FEAT,), _i32),
                       pltpu.VMEM((_BINS_W * _E_FEAT,), _f32),
                       pltpu.VMEM((_BINS_W, _MEM_DIM), _f32),
                       pltpu.VMEM((_BINS_W, _MEM_DIM), _f32),
                       pltpu.SemaphoreType.DMA],
        compiler_params=sc_params,
    )(_merge_and_gather)
    best, segts, delta, nodets, featw_fl, ssrc, sdst = stage_bc(
        pts.reshape(-1), pid.reshape(-1), edge_src, edge_feats.reshape(-1),
        nid_pad, last_update_t, memory)
    featw = featw_fl.reshape(_N_PAD, _E_FEAT)

    B = 512
    grid = (_N_PAD // B,)
    D = _MEM_DIM
    wis = W_ih[:, :D].T
    wid_w = W_ih[:, D:2 * D].T
    wif = W_ih[:, 2 * D:2 * D + _E_FEAT].T
    wit = W_ih[:, 2 * D + _E_FEAT:].T
    whh = W_hh.T
    tw = time_w.reshape(1, _T_DIM)
    tb = time_b.reshape(1, _T_DIM)
    bih = b_ih.reshape(1, 3 * D)
    bhh = b_hh.reshape(1, 3 * D)

    col = lambda a: a.reshape(_N_PAD, 1)
    row_spec = lambda w: pl.BlockSpec((B, w), lambda i: (i, 0))
    full_spec = lambda s: pl.BlockSpec(s, lambda i: (0, 0))

    snew, newts = pl.pallas_call(
        _gru_block,
        grid=grid,
        in_specs=[row_spec(D), row_spec(D), row_spec(_E_FEAT),
                  row_spec(1), row_spec(1), row_spec(1), row_spec(1),
                  full_spec((1, _T_DIM)), full_spec((1, _T_DIM)),
                  full_spec((D, 3 * D)), full_spec((D, 3 * D)),
                  full_spec((_E_FEAT, 3 * D)), full_spec((_T_DIM, 3 * D)),
                  full_spec((D, 3 * D)), full_spec((1, 3 * D)),
                  full_spec((1, 3 * D))],
        out_specs=[row_spec(D), row_spec(1)],
        out_shape=[jax.ShapeDtypeStruct((_N_PAD, D), _f32),
                   jax.ShapeDtypeStruct((_N_PAD, 1), _f32)],
    )(ssrc, sdst, featw, col(delta), col(segts), col(nodets), col(best),
      tw, tb, wis, wid_w, wif, wit, whh, bih, bhh)

    return snew[:_N_NODES], newts[:_N_NODES, 0]


# speedup vs baseline: 31.3135x; 1.3285x over previous
import functools
import jax
import jax.numpy as jnp
from jax import lax
from jax.experimental import pallas as pl
from jax.experimental.pallas import tpu as pltpu
from jax.experimental.pallas import tpu_sc as plsc

_N_NODES = 10000
_N_EDGES = 320000
_MEM_DIM = 128
_E_FEAT = 16
_T_DIM = 100

_NC = 2
_NS = 16
_NW = _NC * _NS
_N_PAD = 10240
_BINS_W = _N_PAD // _NW
_EDGES_W = _N_EDGES // _NW
_L = 16

_f32 = jnp.float32
_i32 = jnp.int32


def _seg_argmax_partial(ts_hbm, dst_hbm, pts_hbm, pid_hbm,
                        ts_v, dst_v, bt_v, be_v):
    wid = lax.axis_index("s") * _NC + lax.axis_index("c")
    ebase = wid * _EDGES_W
    pltpu.sync_copy(ts_hbm.at[pl.ds(ebase, _EDGES_W)], ts_v)
    pltpu.sync_copy(dst_hbm.at[pl.ds(ebase, _EDGES_W)], dst_v)

    neg = jnp.full((_L,), -jnp.inf, _f32)
    mone = jnp.full((_L,), -1, _i32)

    def init_body(i, c):
        bt_v[pl.ds(i * _L, _L)] = neg
        be_v[pl.ds(i * _L, _L)] = mone
        return c
    lax.fori_loop(0, _N_PAD // _L, init_body, 0)

    iota = lax.iota(_i32, _L)

    def edge_body(i, c):
        off = i * _L
        d = dst_v[pl.ds(off, _L)]
        t = ts_v[pl.ds(off, _L)]
        e = iota + (ebase + off)

        at = plsc.load_gather(bt_v, [d])
        ae = plsc.load_gather(be_v, [d])
        win = (t > at) | ((t == at) & (e > ae))
        plsc.store_scatter(bt_v, [d], t, mask=win)
        plsc.store_scatter(be_v, [d], e, mask=win)
        at2 = plsc.load_gather(bt_v, [d])
        ae2 = plsc.load_gather(be_v, [d])
        win2 = (t > at2) | ((t == at2) & (e > ae2))
        n_left = plsc.all_reduce_population_count(win2)

        @pl.when(n_left[0] > 0)
        def _retry():
            def w_body(r, cc):
                rt = plsc.load_gather(bt_v, [d])
                re = plsc.load_gather(be_v, [d])
                w = (t > rt) | ((t == rt) & (e > re))
                plsc.store_scatter(bt_v, [d], t, mask=w)
                plsc.store_scatter(be_v, [d], e, mask=w)
                return cc
            lax.fori_loop(0, _L, w_body, 0)
        return c
    lax.fori_loop(0, _EDGES_W // _L, edge_body, 0)

    pltpu.sync_copy(bt_v, pts_hbm.at[pl.ds(wid * _N_PAD, _N_PAD)])
    pltpu.sync_copy(be_v, pid_hbm.at[pl.ds(wid * _N_PAD, _N_PAD)])


def _merge_and_gather(ptsf_hbm, pidf_hbm, esrc_hbm, efeatf_hbm, nid_hbm,
                      lut_hbm, mem_hbm,
                      best_o, segts_o, delta_o, nodets_o, featw_o,
                      ssrc_o, sdst_o,
                      bt_v, be_v, mt_v, me_v, bs_v, srcw_v, nsrc_v,
                      tsrc_v, nv_v, ntv_v, delta_v, fidx_v, featfl_v,
                      ssrc_v, sdst_v, sem_p, sem_dst, sem_w, sem_f):
    wid = lax.axis_index("s") * _NC + lax.axis_index("c")
    base = wid * _BINS_W
    nv = _BINS_W // _L

    pltpu.sync_copy(nid_hbm.at[pl.ds(base, _BINS_W)], nv_v)
    h_ntv = pltpu.async_copy(lut_hbm.at[nv_v], ntv_v, sem_dst)
    h_sdst = pltpu.async_copy(mem_hbm.at[nv_v], sdst_v, sem_dst)

    def fire_body(t, c):
        o = t * _BINS_W
        g = t * _N_PAD + base
        pltpu.async_copy(ptsf_hbm.at[pl.ds(g, _BINS_W)],
                         mt_v.at[pl.ds(o, _BINS_W)], sem_p)
        pltpu.async_copy(pidf_hbm.at[pl.ds(g, _BINS_W)],
                         me_v.at[pl.ds(o, _BINS_W)], sem_p)
        return c
    lax.fori_loop(0, _NW, fire_body, 0)

    def drain_body(t, c):
        o = t * _BINS_W
        g = t * _N_PAD + base
        pltpu.make_async_copy(ptsf_hbm.at[pl.ds(g, _BINS_W)],
                              mt_v.at[pl.ds(o, _BINS_W)], sem_p).wait()
        pltpu.make_async_copy(pidf_hbm.at[pl.ds(g, _BINS_W)],
                              me_v.at[pl.ds(o, _BINS_W)], sem_p).wait()
        return c
    lax.fori_loop(0, _NW, drain_body, 0)

    def vb_init(v, c):
        o = v * _L
        bt_v[pl.ds(o, _L)] = mt_v[pl.ds(o, _L)]
        be_v[pl.ds(o, _L)] = me_v[pl.ds(o, _L)]
        return c
    lax.fori_loop(0, nv, vb_init, 0)

    def merge_body(t, c):
        def vb(v, cc):
            o = v * _L
            bt = bt_v[pl.ds(o, _L)]
            be = be_v[pl.ds(o, _L)]
            mt = mt_v[pl.ds(t * _BINS_W + o, _L)]
            me = me_v[pl.ds(t * _BINS_W + o, _L)]
            w = (mt > bt) | ((mt == bt) & (me > be))
            bt_v[pl.ds(o, _L)] = jnp.where(w, mt, bt)
            be_v[pl.ds(o, _L)] = jnp.where(w, me, be)
            return cc
        lax.fori_loop(0, nv, vb, 0)
        return c
    lax.fori_loop(1, _NW, merge_body, 0)

    iota = lax.iota(_i32, _L)

    def bs_body(v, c):
        o = v * _L
        bs_vec = jnp.maximum(be_v[pl.ds(o, _L)], 0)
        bs_v[pl.ds(o, _L)] = bs_vec
        pos = iota * _E_FEAT + v * (_L * _E_FEAT)
        val = bs_vec * _E_FEAT
        for j in range(_E_FEAT):
            plsc.store_scatter(fidx_v, [pos + j], val + j)
        return c
    lax.fori_loop(0, nv, bs_body, 0)

    h_srcw = pltpu.async_copy(esrc_hbm.at[bs_v], srcw_v, sem_w)
    h_feat = pltpu.async_copy(efeatf_hbm.at[fidx_v], featfl_v, sem_f)
    h_srcw.wait()
    pltpu.async_copy(nid_hbm.at[srcw_v], nsrc_v, sem_w).wait()
    h_tsrc = pltpu.async_copy(lut_hbm.at[nsrc_v], tsrc_v, sem_w)
    h_ssrc = pltpu.async_copy(mem_hbm.at[nsrc_v], ssrc_v, sem_w)
    h_tsrc.wait()

    def d_body(v, c):
        o = v * _L
        delta_v[pl.ds(o, _L)] = bt_v[pl.ds(o, _L)] - tsrc_v[pl.ds(o, _L)]
        return c
    lax.fori_loop(0, nv, d_body, 0)

    h_ssrc.wait()
    h_ntv.wait()
    h_sdst.wait()
    h_feat.wait()

    pltpu.sync_copy(be_v, best_o.at[pl.ds(base, _BINS_W)])
    pltpu.sync_copy(bt_v, segts_o.at[pl.ds(base, _BINS_W)])
    pltpu.sync_copy(delta_v, delta_o.at[pl.ds(base, _BINS_W)])
    pltpu.sync_copy(ntv_v, nodets_o.at[pl.ds(base, _BINS_W)])
    pltpu.sync_copy(featfl_v, featw_o.at[pl.ds(base * _E_FEAT, _BINS_W * _E_FEAT)])
    pltpu.sync_copy(ssrc_v, ssrc_o.at[pl.ds(base, _BINS_W)])
    pltpu.sync_copy(sdst_v, sdst_o.at[pl.ds(base, _BINS_W)])


def _gru_block(ssrc_ref, sdst_ref, featw_ref, delta_ref, segts_ref,
               nodets_ref, best_ref, tw_ref, tb_ref, wis_ref, wid_ref,
               wif_ref, wit_ref, whh_ref, bih_ref, bhh_ref,
               snew_ref, newts_ref):
    f32 = jnp.float32
    delta = delta_ref[...]
    te = jnp.cos(delta * tw_ref[...] + tb_ref[...])
    sdst = sdst_ref[...]
    gi = (jnp.dot(ssrc_ref[...], wis_ref[...], preferred_element_type=f32)
          + jnp.dot(sdst, wid_ref[...], preferred_element_type=f32)
          + jnp.dot(featw_ref[...], wif_ref[...], preferred_element_type=f32)
          + jnp.dot(te, wit_ref[...], preferred_element_type=f32)
          + bih_ref[...])
    gh = jnp.dot(sdst, whh_ref[...], preferred_element_type=f32) + bhh_ref[...]
    D = _MEM_DIM
    r = jax.nn.sigmoid(gi[:, :D] + gh[:, :D])
    z = jax.nn.sigmoid(gi[:, D:2 * D] + gh[:, D:2 * D])
    n = jnp.tanh(gi[:, 2 * D:] + r * gh[:, 2 * D:])
    s_all = (1.0 - z) * n + z * sdst
    has = best_ref[...] >= 0
    snew_ref[...] = jnp.where(has, s_all, sdst)
    newts_ref[...] = jnp.where(has, segts_ref[...], nodets_ref[...])


def kernel(memory, last_update_t, edge_ts, edge_feats, time_w, time_b,
           W_ih, W_hh, b_ih, b_hh, nid, edge_src, edge_dst):
    nid = nid.astype(_i32)
    edge_src = edge_src.astype(_i32)
    edge_dst = edge_dst.astype(_i32)
    nid_pad = jnp.pad(nid, (0, _N_PAD - _N_NODES))

    mesh = plsc.VectorSubcoreMesh(core_axis_name="c", subcore_axis_name="s")
    sc_params = pltpu.CompilerParams(needs_layout_passes=False)

    stage_a = functools.partial(
        pl.kernel,
        out_type=(jax.ShapeDtypeStruct((_NW * _N_PAD,), _f32),
                  jax.ShapeDtypeStruct((_NW * _N_PAD,), _i32)),
        mesh=mesh,
        scratch_types=[pltpu.VMEM((_EDGES_W,), _f32),
                       pltpu.VMEM((_EDGES_W,), _i32),
                       pltpu.VMEM((_N_PAD,), _f32),
                       pltpu.VMEM((_N_PAD,), _i32)],
        compiler_params=sc_params,
    )(_seg_argmax_partial)
    pts, pid = stage_a(edge_ts, edge_dst)

    stage_bc = functools.partial(
        pl.kernel,
        out_type=(jax.ShapeDtypeStruct((_N_PAD,), _i32),
                  jax.ShapeDtypeStruct((_N_PAD,), _f32),
                  jax.ShapeDtypeStruct((_N_PAD,), _f32),
                  jax.ShapeDtypeStruct((_N_PAD,), _f32),
                  jax.ShapeDtypeStruct((_N_PAD * _E_FEAT,), _f32),
                  jax.ShapeDtypeStruct((_N_PAD, _MEM_DIM), _f32),
                  jax.ShapeDtypeStruct((_N_PAD, _MEM_DIM), _f32)),
        mesh=mesh,
        scratch_types=[pltpu.VMEM((_BINS_W,), _f32),
                       pltpu.VMEM((_BINS_W,), _i32),
                       pltpu.VMEM((_NW * _BINS_W,), _f32),
                       pltpu.VMEM((_NW * _BINS_W,), _i32),
                       pltpu.VMEM((_BINS_W,), _i32),
                       pltpu.VMEM((_BINS_W,), _i32),
                       pltpu.VMEM((_BINS_W,), _i32),
                       pltpu.VMEM((_BINS_W,), _f32),
                       pltpu.VMEM((_BINS_W,), _i32),
                       pltpu.VMEM((_BINS_W,), _f32),
                       pltpu.VMEM((_BINS_W,), _f32),
                       pltpu.VMEM((_BINS_W * _E_FEAT,), _i32),
                       pltpu.VMEM((_BINS_W * _E_FEAT,), _f32),
                       pltpu.VMEM((_BINS_W, _MEM_DIM), _f32),
                       pltpu.VMEM((_BINS_W, _MEM_DIM), _f32),
                       pltpu.SemaphoreType.DMA,
                       pltpu.SemaphoreType.DMA,
                       pltpu.SemaphoreType.DMA,
                       pltpu.SemaphoreType.DMA],
        compiler_params=sc_params,
    )(_merge_and_gather)
    best, segts, delta, nodets, featw_fl, ssrc, sdst = stage_bc(
        pts, pid, edge_src, edge_feats.reshape(-1),
        nid_pad, last_update_t, memory)
    featw = featw_fl.reshape(_N_PAD, _E_FEAT)

    B = 512
    grid = (_N_PAD // B,)
    D = _MEM_DIM
    wis = W_ih[:, :D].T
    wid_w = W_ih[:, D:2 * D].T
    wif = W_ih[:, 2 * D:2 * D + _E_FEAT].T
    wit = W_ih[:, 2 * D + _E_FEAT:].T
    whh = W_hh.T
    tw = time_w.reshape(1, _T_DIM)
    tb = time_b.reshape(1, _T_DIM)
    bih = b_ih.reshape(1, 3 * D)
    bhh = b_hh.reshape(1, 3 * D)

    col = lambda a: a.reshape(_N_PAD, 1)
    row_spec = lambda w: pl.BlockSpec((B, w), lambda i: (i, 0))
    full_spec = lambda s: pl.BlockSpec(s, lambda i: (0, 0))

    snew, newts = pl.pallas_call(
        _gru_block,
        grid=grid,
        in_specs=[row_spec(D), row_spec(D), row_spec(_E_FEAT),
                  row_spec(1), row_spec(1), row_spec(1), row_spec(1),
                  full_spec((1, _T_DIM)), full_spec((1, _T_DIM)),
                  full_spec((D, 3 * D)), full_spec((D, 3 * D)),
                  full_spec((_E_FEAT, 3 * D)), full_spec((_T_DIM, 3 * D)),
                  full_spec((D, 3 * D)), full_spec((1, 3 * D)),
                  full_spec((1, 3 * D))],
        out_specs=[row_spec(D), row_spec(1)],
        out_shape=[jax.ShapeDtypeStruct((_N_PAD, D), _f32),
                   jax.ShapeDtypeStruct((_N_PAD, 1), _f32)],
    )(ssrc, sdst, featw, col(delta), col(segts), col(nodets), col(best),
      tw, tb, wis, wid_w, wif, wit, whh, bih, bhh)

    return snew[:_N_NODES], newts[:_N_NODES, 0]


# speedup vs baseline: 34.8204x; 1.1120x over previous
import functools
import jax
import jax.numpy as jnp
from jax import lax
from jax.experimental import pallas as pl
from jax.experimental.pallas import tpu as pltpu
from jax.experimental.pallas import tpu_sc as plsc

_N_NODES = 10000
_N_EDGES = 320000
_MEM_DIM = 128
_E_FEAT = 16
_T_DIM = 100

_NC = 2
_NS = 16
_NW = _NC * _NS
_N_PAD = 10240
_BINS_W = _N_PAD // _NW
_EDGES_W = _N_EDGES // _NW
_L = 16

_f32 = jnp.float32
_i32 = jnp.int32


def _seg_argmax_partial(ts_hbm, dst_hbm, pts_hbm, pid_hbm,
                        ts_v, dst_v, bt_v, be_v):
    wid = lax.axis_index("s") * _NC + lax.axis_index("c")
    ebase = wid * _EDGES_W
    pltpu.sync_copy(ts_hbm.at[pl.ds(ebase, _EDGES_W)], ts_v)
    pltpu.sync_copy(dst_hbm.at[pl.ds(ebase, _EDGES_W)], dst_v)

    neg = jnp.full((_L,), -jnp.inf, _f32)
    mone = jnp.full((_L,), -1, _i32)

    def init_body(i, c):
        bt_v[pl.ds(i * _L, _L)] = neg
        be_v[pl.ds(i * _L, _L)] = mone
        return c
    lax.fori_loop(0, _N_PAD // _L, init_body, 0)

    iota = lax.iota(_i32, _L)

    def edge_body(i, c):
        off = i * _L
        d = dst_v[pl.ds(off, _L)]
        t = ts_v[pl.ds(off, _L)]
        e = iota + (ebase + off)

        at = plsc.load_gather(bt_v, [d])
        ae = plsc.load_gather(be_v, [d])
        win = (t > at) | ((t == at) & (e > ae))
        plsc.store_scatter(bt_v, [d], t, mask=win)
        plsc.store_scatter(be_v, [d], e, mask=win)
        at2 = plsc.load_gather(bt_v, [d])
        ae2 = plsc.load_gather(be_v, [d])
        win2 = (t > at2) | ((t == at2) & (e > ae2))
        n_left = plsc.all_reduce_population_count(win2)

        @pl.when(n_left[0] > 0)
        def _retry():
            def w_body(r, cc):
                rt = plsc.load_gather(bt_v, [d])
                re = plsc.load_gather(be_v, [d])
                w = (t > rt) | ((t == rt) & (e > re))
                plsc.store_scatter(bt_v, [d], t, mask=w)
                plsc.store_scatter(be_v, [d], e, mask=w)
                return cc
            lax.fori_loop(0, _L, w_body, 0)
        return c
    lax.fori_loop(0, _EDGES_W // _L, edge_body, 0)

    pltpu.sync_copy(bt_v, pts_hbm.at[pl.ds(wid * _N_PAD, _N_PAD)])
    pltpu.sync_copy(be_v, pid_hbm.at[pl.ds(wid * _N_PAD, _N_PAD)])


def _merge_and_gather(ptsf_hbm, pidf_hbm, esrc_hbm, efeat_hbm, nid_hbm,
                      lut_hbm, mem_hbm,
                      newts_o, delta_o, featw_o, ssrc_o, sdst_o,
                      bt_v, be_v, mt_v, me_v, bs_v, srcw_v, nsrc_v,
                      tsrc_v, nv_v, ntv_v, newts_v, delta_v, featw_v,
                      ssrc_v, sdst_v, sem_p, sem_dst, sem_w, sem_f):
    wid = lax.axis_index("s") * _NC + lax.axis_index("c")
    base = wid * _BINS_W
    nv = _BINS_W // _L

    pltpu.sync_copy(nid_hbm.at[pl.ds(base, _BINS_W)], nv_v)
    h_ntv = pltpu.async_copy(lut_hbm.at[nv_v], ntv_v, sem_dst)
    h_sdst = pltpu.async_copy(mem_hbm.at[nv_v], sdst_v, sem_dst)

    def fire_body(t, c):
        o = t * _BINS_W
        g = t * _N_PAD + base
        pltpu.async_copy(ptsf_hbm.at[pl.ds(g, _BINS_W)],
                         mt_v.at[pl.ds(o, _BINS_W)], sem_p)
        pltpu.async_copy(pidf_hbm.at[pl.ds(g, _BINS_W)],
                         me_v.at[pl.ds(o, _BINS_W)], sem_p)
        return c
    lax.fori_loop(0, _NW, fire_body, 0)

    def drain_body(t, c):
        o = t * _BINS_W
        g = t * _N_PAD + base
        pltpu.make_async_copy(ptsf_hbm.at[pl.ds(g, _BINS_W)],
                              mt_v.at[pl.ds(o, _BINS_W)], sem_p).wait()
        pltpu.make_async_copy(pidf_hbm.at[pl.ds(g, _BINS_W)],
                              me_v.at[pl.ds(o, _BINS_W)], sem_p).wait()
        return c
    lax.fori_loop(0, _NW, drain_body, 0)

    def vb_init(v, c):
        o = v * _L
        bt_v[pl.ds(o, _L)] = mt_v[pl.ds(o, _L)]
        be_v[pl.ds(o, _L)] = me_v[pl.ds(o, _L)]
        return c
    lax.fori_loop(0, nv, vb_init, 0)

    def merge_body(t, c):
        def vb(v, cc):
            o = v * _L
            bt = bt_v[pl.ds(o, _L)]
            be = be_v[pl.ds(o, _L)]
            mt = mt_v[pl.ds(t * _BINS_W + o, _L)]
            me = me_v[pl.ds(t * _BINS_W + o, _L)]
            w = (mt > bt) | ((mt == bt) & (me > be))
            bt_v[pl.ds(o, _L)] = jnp.where(w, mt, bt)
            be_v[pl.ds(o, _L)] = jnp.where(w, me, be)
            return cc
        lax.fori_loop(0, nv, vb, 0)
        return c
    lax.fori_loop(1, _NW, merge_body, 0)

    def bs_body(v, c):
        o = v * _L
        bs_v[pl.ds(o, _L)] = jnp.maximum(be_v[pl.ds(o, _L)], 0)
        return c
    lax.fori_loop(0, nv, bs_body, 0)

    h_srcw = pltpu.async_copy(esrc_hbm.at[bs_v], srcw_v, sem_w)
    h_feat = pltpu.async_copy(efeat_hbm.at[bs_v], featw_v, sem_f)
    h_srcw.wait()
    pltpu.async_copy(nid_hbm.at[srcw_v], nsrc_v, sem_w).wait()
    h_tsrc = pltpu.async_copy(lut_hbm.at[nsrc_v], tsrc_v, sem_w)
    h_ssrc = pltpu.async_copy(mem_hbm.at[nsrc_v], ssrc_v, sem_w)
    h_tsrc.wait()
    h_ntv.wait()

    neg = jnp.full((_L,), -jnp.inf, _f32)

    def d_body(v, c):
        o = v * _L
        bt = bt_v[pl.ds(o, _L)]
        delta_v[pl.ds(o, _L)] = bt - tsrc_v[pl.ds(o, _L)]
        newts_v[pl.ds(o, _L)] = jnp.where(bt == neg, ntv_v[pl.ds(o, _L)], bt)
        return c
    lax.fori_loop(0, nv, d_body, 0)

    h_ssrc.wait()
    h_sdst.wait()
    h_feat.wait()

    pltpu.sync_copy(newts_v, newts_o.at[pl.ds(base, _BINS_W)])
    pltpu.sync_copy(delta_v, delta_o.at[pl.ds(base, _BINS_W)])
    pltpu.sync_copy(featw_v, featw_o.at[pl.ds(base, _BINS_W)])
    pltpu.sync_copy(ssrc_v, ssrc_o.at[pl.ds(base, _BINS_W)])
    pltpu.sync_copy(sdst_v, sdst_o.at[pl.ds(base, _BINS_W)])


def _gru_block(ssrc_ref, sdst_ref, featw_ref, delta_ref,
               tw_ref, tb_ref, wis_ref, wid_ref,
               wif_ref, wit_ref, whh_ref, bih_ref, bhh_ref,
               snew_ref):
    f32 = jnp.float32
    delta = delta_ref[...]
    te = jnp.cos(delta * tw_ref[...] + tb_ref[...])
    sdst = sdst_ref[...]
    gi = (jnp.dot(ssrc_ref[...], wis_ref[...], preferred_element_type=f32)
          + jnp.dot(sdst, wid_ref[...], preferred_element_type=f32)
          + jnp.dot(featw_ref[...], wif_ref[...], preferred_element_type=f32)
          + jnp.dot(te, wit_ref[...], preferred_element_type=f32)
          + bih_ref[...])
    gh = jnp.dot(sdst, whh_ref[...], preferred_element_type=f32) + bhh_ref[...]
    D = _MEM_DIM
    r = jax.nn.sigmoid(gi[:, :D] + gh[:, :D])
    z = jax.nn.sigmoid(gi[:, D:2 * D] + gh[:, D:2 * D])
    n = jnp.tanh(gi[:, 2 * D:] + r * gh[:, 2 * D:])
    s_all = (1.0 - z) * n + z * sdst
    has = delta != -jnp.inf
    snew_ref[...] = jnp.where(has, s_all, sdst)


def kernel(memory, last_update_t, edge_ts, edge_feats, time_w, time_b,
           W_ih, W_hh, b_ih, b_hh, nid, edge_src, edge_dst):
    nid = nid.astype(_i32)
    edge_src = edge_src.astype(_i32)
    edge_dst = edge_dst.astype(_i32)
    nid_pad = jnp.pad(nid, (0, _N_PAD - _N_NODES))

    mesh = plsc.VectorSubcoreMesh(core_axis_name="c", subcore_axis_name="s")
    sc_params = pltpu.CompilerParams(needs_layout_passes=False)

    stage_a = functools.partial(
        pl.kernel,
        out_type=(jax.ShapeDtypeStruct((_NW * _N_PAD,), _f32),
                  jax.ShapeDtypeStruct((_NW * _N_PAD,), _i32)),
        mesh=mesh,
        scratch_types=[pltpu.VMEM((_EDGES_W,), _f32),
                       pltpu.VMEM((_EDGES_W,), _i32),
                       pltpu.VMEM((_N_PAD,), _f32),
                       pltpu.VMEM((_N_PAD,), _i32)],
        compiler_params=sc_params,
    )(_seg_argmax_partial)
    pts, pid = stage_a(edge_ts, edge_dst)

    stage_bc = functools.partial(
        pl.kernel,
        out_type=(jax.ShapeDtypeStruct((_N_PAD,), _f32),
                  jax.ShapeDtypeStruct((_N_PAD,), _f32),
                  jax.ShapeDtypeStruct((_N_PAD, _E_FEAT), _f32),
                  jax.ShapeDtypeStruct((_N_PAD, _MEM_DIM), _f32),
                  jax.ShapeDtypeStruct((_N_PAD, _MEM_DIM), _f32)),
        mesh=mesh,
        scratch_types=[pltpu.VMEM((_BINS_W,), _f32),
                       pltpu.VMEM((_BINS_W,), _i32),
                       pltpu.VMEM((_NW * _BINS_W,), _f32),
                       pltpu.VMEM((_NW * _BINS_W,), _i32),
                       pltpu.VMEM((_BINS_W,), _i32),
                       pltpu.VMEM((_BINS_W,), _i32),
                       pltpu.VMEM((_BINS_W,), _i32),
                       pltpu.VMEM((_BINS_W,), _f32),
                       pltpu.VMEM((_BINS_W,), _i32),
                       pltpu.VMEM((_BINS_W,), _f32),
                       pltpu.VMEM((_BINS_W,), _f32),
                       pltpu.VMEM((_BINS_W,), _f32),
                       pltpu.VMEM((_BINS_W, _E_FEAT), _f32),
                       pltpu.VMEM((_BINS_W, _MEM_DIM), _f32),
                       pltpu.VMEM((_BINS_W, _MEM_DIM), _f32),
                       pltpu.SemaphoreType.DMA,
                       pltpu.SemaphoreType.DMA,
                       pltpu.SemaphoreType.DMA,
                       pltpu.SemaphoreType.DMA],
        compiler_params=pltpu.CompilerParams(needs_layout_passes=False,
                                             use_tc_tiling_on_sc=False),
    )(_merge_and_gather)
    newts, delta, featw, ssrc, sdst = stage_bc(
        pts, pid, edge_src, edge_feats,
        nid_pad, last_update_t, memory)

    B = 512
    grid = (_N_PAD // B,)
    D = _MEM_DIM
    wis = W_ih[:, :D].T
    wid_w = W_ih[:, D:2 * D].T
    wif = W_ih[:, 2 * D:2 * D + _E_FEAT].T
    wit = W_ih[:, 2 * D + _E_FEAT:].T
    whh = W_hh.T
    tw = time_w.reshape(1, _T_DIM)
    tb = time_b.reshape(1, _T_DIM)
    bih = b_ih.reshape(1, 3 * D)
    bhh = b_hh.reshape(1, 3 * D)

    col = lambda a: a.reshape(_N_PAD, 1)
    row_spec = lambda w: pl.BlockSpec((B, w), lambda i: (i, 0))
    full_spec = lambda s: pl.BlockSpec(s, lambda i: (0, 0))

    snew = pl.pallas_call(
        _gru_block,
        grid=grid,
        in_specs=[row_spec(D), row_spec(D), row_spec(_E_FEAT),
                  row_spec(1),
                  full_spec((1, _T_DIM)), full_spec((1, _T_DIM)),
                  full_spec((D, 3 * D)), full_spec((D, 3 * D)),
                  full_spec((_E_FEAT, 3 * D)), full_spec((_T_DIM, 3 * D)),
                  full_spec((D, 3 * D)), full_spec((1, 3 * D)),
                  full_spec((1, 3 * D))],
        out_specs=pl.BlockSpec((B, D), lambda i: (i, 0)),
        out_shape=jax.ShapeDtypeStruct((_N_PAD, D), _f32),
    )(ssrc, sdst, featw, col(delta),
      tw, tb, wis, wid_w, wif, wit, whh, bih, bhh)

    return snew[:_N_NODES], newts[:_N_NODES]


# speedup vs baseline: 51.6265x; 1.4827x over previous
import functools
import jax
import jax.numpy as jnp
from jax import lax
from jax.experimental import pallas as pl
from jax.experimental.pallas import tpu as pltpu
from jax.experimental.pallas import tpu_sc as plsc

_N_NODES = 10000
_N_EDGES = 320000
_MEM_DIM = 128
_E_FEAT = 16
_T_DIM = 100

_NC = 2
_NS = 16
_NW = _NC * _NS
_N_PAD = 10240
_BINS_W = _N_PAD // _NW
_EDGES_W = _N_EDGES // _NW
_L = 16

_f32 = jnp.float32
_i32 = jnp.int32


def _seg_argmax_partial(ts_hbm, dst_hbm, pts_hbm, pid_hbm,
                        ts_v, dst_v, bt_v, be_v):
    wid = lax.axis_index("s") * _NC + lax.axis_index("c")
    ebase = wid * _EDGES_W
    pltpu.sync_copy(ts_hbm.at[pl.ds(ebase, _EDGES_W)], ts_v)
    pltpu.sync_copy(dst_hbm.at[pl.ds(ebase, _EDGES_W)], dst_v)

    neg = jnp.full((_L,), -jnp.inf, _f32)
    mone = jnp.full((_L,), -1, _i32)

    def init_body(i, c):
        bt_v[pl.ds(i * _L, _L)] = neg
        be_v[pl.ds(i * _L, _L)] = mone
        return c
    lax.fori_loop(0, _N_PAD // _L, init_body, 0)

    iota = lax.iota(_i32, _L)

    def edge_body(i, c):
        off = i * _L
        d = dst_v[pl.ds(off, _L)]
        t = ts_v[pl.ds(off, _L)]
        e = iota + (ebase + off)

        at = plsc.load_gather(bt_v, [d])
        ae = plsc.load_gather(be_v, [d])
        win = (t > at) | ((t == at) & (e > ae))
        plsc.store_scatter(bt_v, [d], t, mask=win)
        plsc.store_scatter(be_v, [d], e, mask=win)
        at2 = plsc.load_gather(bt_v, [d])
        ae2 = plsc.load_gather(be_v, [d])
        win2 = (t > at2) | ((t == at2) & (e > ae2))
        n_left = plsc.all_reduce_population_count(win2)

        @pl.when(n_left[0] > 0)
        def _retry():
            def w_body(r, cc):
                rt = plsc.load_gather(bt_v, [d])
                re = plsc.load_gather(be_v, [d])
                w = (t > rt) | ((t == rt) & (e > re))
                plsc.store_scatter(bt_v, [d], t, mask=w)
                plsc.store_scatter(be_v, [d], e, mask=w)
                return cc
            lax.fori_loop(0, _L, w_body, 0)
        return c
    lax.fori_loop(0, _EDGES_W // _L, edge_body, 0)

    pltpu.sync_copy(bt_v, pts_hbm.at[pl.ds(wid * _N_PAD, _N_PAD)])
    pltpu.sync_copy(be_v, pid_hbm.at[pl.ds(wid * _N_PAD, _N_PAD)])


def _merge_and_gather(ptsf_hbm, pidf_hbm, esrc_hbm, efeat_hbm, nid_hbm,
                      lut_hbm, mem_hbm,
                      newts_o, delta_o, featw_o, ssrc_o, sdst_o,
                      bt_v, be_v, mt_v, me_v, bs_v, srcw_v, nsrc_v,
                      tsrc_v, nv_v, ntv_v, newts_v, delta_v, fidx_v,
                      featwT_v, ssrc_v, sdst_v, sem_p, sem_dst, sem_w,
                      sem_f):
    wid = lax.axis_index("s") * _NC + lax.axis_index("c")
    base = wid * _BINS_W
    nv = _BINS_W // _L

    pltpu.sync_copy(nid_hbm.at[pl.ds(base, _BINS_W)], nv_v)
    h_ntv = pltpu.async_copy(lut_hbm.at[nv_v], ntv_v, sem_dst)
    h_sdst = pltpu.async_copy(mem_hbm.at[nv_v], sdst_v, sem_dst)

    def fire_body(t, c):
        o = t * _BINS_W
        g = t * _N_PAD + base
        pltpu.async_copy(ptsf_hbm.at[pl.ds(g, _BINS_W)],
                         mt_v.at[pl.ds(o, _BINS_W)], sem_p)
        pltpu.async_copy(pidf_hbm.at[pl.ds(g, _BINS_W)],
                         me_v.at[pl.ds(o, _BINS_W)], sem_p)
        return c
    lax.fori_loop(0, _NW, fire_body, 0)

    def drain_body(t, c):
        o = t * _BINS_W
        g = t * _N_PAD + base
        pltpu.make_async_copy(ptsf_hbm.at[pl.ds(g, _BINS_W)],
                              mt_v.at[pl.ds(o, _BINS_W)], sem_p).wait()
        pltpu.make_async_copy(pidf_hbm.at[pl.ds(g, _BINS_W)],
                              me_v.at[pl.ds(o, _BINS_W)], sem_p).wait()
        return c
    lax.fori_loop(0, _NW, drain_body, 0)

    def vb_init(v, c):
        o = v * _L
        bt_v[pl.ds(o, _L)] = mt_v[pl.ds(o, _L)]
        be_v[pl.ds(o, _L)] = me_v[pl.ds(o, _L)]
        return c
    lax.fori_loop(0, nv, vb_init, 0)

    def merge_body(t, c):
        def vb(v, cc):
            o = v * _L
            bt = bt_v[pl.ds(o, _L)]
            be = be_v[pl.ds(o, _L)]
            mt = mt_v[pl.ds(t * _BINS_W + o, _L)]
            me = me_v[pl.ds(t * _BINS_W + o, _L)]
            w = (mt > bt) | ((mt == bt) & (me > be))
            bt_v[pl.ds(o, _L)] = jnp.where(w, mt, bt)
            be_v[pl.ds(o, _L)] = jnp.where(w, me, be)
            return cc
        lax.fori_loop(0, nv, vb, 0)
        return c
    lax.fori_loop(1, _NW, merge_body, 0)

    def bs_body(v, c):
        o = v * _L
        bs_vec = jnp.maximum(be_v[pl.ds(o, _L)], 0)
        bs_v[pl.ds(o, _L)] = bs_vec
        for j in range(_E_FEAT):
            fidx_v[pl.ds(j * _BINS_W + o, _L)] = bs_vec + j * _N_EDGES
        return c
    lax.fori_loop(0, nv, bs_body, 0)

    h_srcw = pltpu.async_copy(esrc_hbm.at[bs_v], srcw_v, sem_w)
    for j in range(_E_FEAT):
        pltpu.async_copy(efeat_hbm.at[fidx_v.at[pl.ds(j * _BINS_W, _BINS_W)]],
                         featwT_v.at[pl.ds(j * _BINS_W, _BINS_W)], sem_f)
    h_srcw.wait()
    pltpu.async_copy(nid_hbm.at[srcw_v], nsrc_v, sem_w).wait()
    h_tsrc = pltpu.async_copy(lut_hbm.at[nsrc_v], tsrc_v, sem_w)
    h_ssrc = pltpu.async_copy(mem_hbm.at[nsrc_v], ssrc_v, sem_w)
    h_tsrc.wait()
    h_ntv.wait()

    neg = jnp.full((_L,), -jnp.inf, _f32)

    def d_body(v, c):
        o = v * _L
        bt = bt_v[pl.ds(o, _L)]
        delta_v[pl.ds(o, _L)] = bt - tsrc_v[pl.ds(o, _L)]
        newts_v[pl.ds(o, _L)] = jnp.where(bt == neg, ntv_v[pl.ds(o, _L)], bt)
        return c
    lax.fori_loop(0, nv, d_body, 0)

    h_ssrc.wait()
    h_sdst.wait()
    for j in range(_E_FEAT):
        pltpu.make_async_copy(
            efeat_hbm.at[fidx_v.at[pl.ds(j * _BINS_W, _BINS_W)]],
            featwT_v.at[pl.ds(j * _BINS_W, _BINS_W)], sem_f).wait()

    pltpu.sync_copy(newts_v, newts_o.at[pl.ds(base, _BINS_W)])
    pltpu.sync_copy(delta_v, delta_o.at[pl.ds(base, _BINS_W)])
    for j in range(_E_FEAT):
        pltpu.sync_copy(featwT_v.at[pl.ds(j * _BINS_W, _BINS_W)],
                        featw_o.at[pl.ds(j * _N_PAD + base, _BINS_W)])
    pltpu.sync_copy(ssrc_v, ssrc_o.at[pl.ds(base, _BINS_W)])
    pltpu.sync_copy(sdst_v, sdst_o.at[pl.ds(base, _BINS_W)])


def _gru_block(ssrc_ref, sdst_ref, featw_ref, delta_ref,
               tw_ref, tb_ref, wis_ref, wid_ref,
               wif_ref, wit_ref, whh_ref, bih_ref, bhh_ref,
               snew_ref):
    f32 = jnp.float32
    delta = delta_ref[...]
    te = jnp.cos(delta * tw_ref[...] + tb_ref[...])
    sdst = sdst_ref[...]
    gi = (jnp.dot(ssrc_ref[...], wis_ref[...], preferred_element_type=f32)
          + jnp.dot(sdst, wid_ref[...], preferred_element_type=f32)
          + lax.dot_general(featw_ref[...], wif_ref[...],
                            (((0,), (0,)), ((), ())),
                            preferred_element_type=f32)
          + jnp.dot(te, wit_ref[...], preferred_element_type=f32)
          + bih_ref[...])
    gh = jnp.dot(sdst, whh_ref[...], preferred_element_type=f32) + bhh_ref[...]
    D = _MEM_DIM
    r = jax.nn.sigmoid(gi[:, :D] + gh[:, :D])
    z = jax.nn.sigmoid(gi[:, D:2 * D] + gh[:, D:2 * D])
    n = jnp.tanh(gi[:, 2 * D:] + r * gh[:, 2 * D:])
    s_all = (1.0 - z) * n + z * sdst
    has = delta != -jnp.inf
    snew_ref[...] = jnp.where(has, s_all, sdst)


def kernel(memory, last_update_t, edge_ts, edge_feats, time_w, time_b,
           W_ih, W_hh, b_ih, b_hh, nid, edge_src, edge_dst):
    nid = nid.astype(_i32)
    edge_src = edge_src.astype(_i32)
    edge_dst = edge_dst.astype(_i32)
    nid_pad = jnp.pad(nid, (0, _N_PAD - _N_NODES))

    mesh = plsc.VectorSubcoreMesh(core_axis_name="c", subcore_axis_name="s")
    sc_params = pltpu.CompilerParams(needs_layout_passes=False)

    stage_a = functools.partial(
        pl.kernel,
        out_type=(jax.ShapeDtypeStruct((_NW * _N_PAD,), _f32),
                  jax.ShapeDtypeStruct((_NW * _N_PAD,), _i32)),
        mesh=mesh,
        scratch_types=[pltpu.VMEM((_EDGES_W,), _f32),
                       pltpu.VMEM((_EDGES_W,), _i32),
                       pltpu.VMEM((_N_PAD,), _f32),
                       pltpu.VMEM((_N_PAD,), _i32)],
        compiler_params=sc_params,
    )(_seg_argmax_partial)
    pts, pid = stage_a(edge_ts, edge_dst)

    stage_bc = functools.partial(
        pl.kernel,
        out_type=(jax.ShapeDtypeStruct((_N_PAD,), _f32),
                  jax.ShapeDtypeStruct((_N_PAD,), _f32),
                  jax.ShapeDtypeStruct((_E_FEAT * _N_PAD,), _f32),
                  jax.ShapeDtypeStruct((_N_PAD, _MEM_DIM), _f32),
                  jax.ShapeDtypeStruct((_N_PAD, _MEM_DIM), _f32)),
        mesh=mesh,
        scratch_types=[pltpu.VMEM((_BINS_W,), _f32),
                       pltpu.VMEM((_BINS_W,), _i32),
                       pltpu.VMEM((_NW * _BINS_W,), _f32),
                       pltpu.VMEM((_NW * _BINS_W,), _i32),
                       pltpu.VMEM((_BINS_W,), _i32),
                       pltpu.VMEM((_BINS_W,), _i32),
                       pltpu.VMEM((_BINS_W,), _i32),
                       pltpu.VMEM((_BINS_W,), _f32),
                       pltpu.VMEM((_BINS_W,), _i32),
                       pltpu.VMEM((_BINS_W,), _f32),
                       pltpu.VMEM((_BINS_W,), _f32),
                       pltpu.VMEM((_BINS_W,), _f32),
                       pltpu.VMEM((_E_FEAT * _BINS_W,), _i32),
                       pltpu.VMEM((_E_FEAT * _BINS_W,), _f32),
                       pltpu.VMEM((_BINS_W, _MEM_DIM), _f32),
                       pltpu.VMEM((_BINS_W, _MEM_DIM), _f32),
                       pltpu.SemaphoreType.DMA,
                       pltpu.SemaphoreType.DMA,
                       pltpu.SemaphoreType.DMA,
                       pltpu.SemaphoreType.DMA],
        compiler_params=pltpu.CompilerParams(needs_layout_passes=False,
                                             use_tc_tiling_on_sc=False),
    )(_merge_and_gather)
    newts, delta, featw_fl, ssrc, sdst = stage_bc(
        pts, pid, edge_src, edge_feats.T.reshape(-1),
        nid_pad, last_update_t, memory)
    featw = featw_fl.reshape(_E_FEAT, _N_PAD)

    B = 512
    grid = (_N_PAD // B,)
    D = _MEM_DIM
    wis = W_ih[:, :D].T
    wid_w = W_ih[:, D:2 * D].T
    wif = W_ih[:, 2 * D:2 * D + _E_FEAT].T
    wit = W_ih[:, 2 * D + _E_FEAT:].T
    whh = W_hh.T
    tw = time_w.reshape(1, _T_DIM)
    tb = time_b.reshape(1, _T_DIM)
    bih = b_ih.reshape(1, 3 * D)
    bhh = b_hh.reshape(1, 3 * D)

    col = lambda a: a.reshape(_N_PAD, 1)
    row_spec = lambda w: pl.BlockSpec((B, w), lambda i: (i, 0))
    full_spec = lambda s: pl.BlockSpec(s, lambda i: (0, 0))

    snew = pl.pallas_call(
        _gru_block,
        grid=grid,
        in_specs=[row_spec(D), row_spec(D),
                  pl.BlockSpec((_E_FEAT, B), lambda i: (0, i)),
                  row_spec(1),
                  full_spec((1, _T_DIM)), full_spec((1, _T_DIM)),
                  full_spec((D, 3 * D)), full_spec((D, 3 * D)),
                  full_spec((_E_FEAT, 3 * D)), full_spec((_T_DIM, 3 * D)),
                  full_spec((D, 3 * D)), full_spec((1, 3 * D)),
                  full_spec((1, 3 * D))],
        out_specs=pl.BlockSpec((B, D), lambda i: (i, 0)),
        out_shape=jax.ShapeDtypeStruct((_N_PAD, D), _f32),
    )(ssrc, sdst, featw, col(delta),
      tw, tb, wis, wid_w, wif, wit, whh, bih, bhh)

    return snew[:_N_NODES], newts[:_N_NODES]


# speedup vs baseline: 55.9749x; 1.0842x over previous
import functools
import jax
import jax.numpy as jnp
from jax import lax
from jax.experimental import pallas as pl
from jax.experimental.pallas import tpu as pltpu
from jax.experimental.pallas import tpu_sc as plsc

_N_NODES = 10000
_N_EDGES = 320000
_MEM_TOTAL = 100000
_MEM_DIM = 128
_E_FEAT = 16
_T_DIM = 100

_NC = 2
_NS = 16
_NW = _NC * _NS
_N_PAD = 10240
_BINS_W = _N_PAD // _NW
_EDGES_W = _N_EDGES // _NW
_L = 16

_f32 = jnp.float32
_i32 = jnp.int32


def _seg_argmax_partial(ts_hbm, dst_hbm, pts_hbm, pid_hbm,
                        ts_v, dst_v, bt_v, be_v):
    wid = lax.axis_index("s") * _NC + lax.axis_index("c")
    ebase = wid * _EDGES_W
    pltpu.sync_copy(ts_hbm.at[pl.ds(ebase, _EDGES_W)], ts_v)
    pltpu.sync_copy(dst_hbm.at[pl.ds(ebase, _EDGES_W)], dst_v)

    neg = jnp.full((_L,), -jnp.inf, _f32)
    mone = jnp.full((_L,), -1, _i32)

    def init_body(i, c):
        bt_v[pl.ds(i * _L, _L)] = neg
        be_v[pl.ds(i * _L, _L)] = mone
        return c
    lax.fori_loop(0, _N_PAD // _L, init_body, 0)

    iota = lax.iota(_i32, _L)

    def edge_body(i, c):
        off = i * _L
        d = dst_v[pl.ds(off, _L)]
        t = ts_v[pl.ds(off, _L)]
        e = iota + (ebase + off)

        at = plsc.load_gather(bt_v, [d])
        ae = plsc.load_gather(be_v, [d])
        win = (t > at) | ((t == at) & (e > ae))
        plsc.store_scatter(bt_v, [d], t, mask=win)
        plsc.store_scatter(be_v, [d], e, mask=win)
        at2 = plsc.load_gather(bt_v, [d])
        ae2 = plsc.load_gather(be_v, [d])
        win2 = (t > at2) | ((t == at2) & (e > ae2))
        n_left = plsc.all_reduce_population_count(win2)

        @pl.when(n_left[0] > 0)
        def _retry():
            def w_body(r, cc):
                rt = plsc.load_gather(bt_v, [d])
                re = plsc.load_gather(be_v, [d])
                w = (t > rt) | ((t == rt) & (e > re))
                plsc.store_scatter(bt_v, [d], t, mask=w)
                plsc.store_scatter(be_v, [d], e, mask=w)
                return cc
            lax.fori_loop(0, _L, w_body, 0)
        return c
    lax.fori_loop(0, _EDGES_W // _L, edge_body, 0)

    pltpu.sync_copy(bt_v, pts_hbm.at[pl.ds(wid * _N_PAD, _N_PAD)])
    pltpu.sync_copy(be_v, pid_hbm.at[pl.ds(wid * _N_PAD, _N_PAD)])


def _merge_and_gather(ptsf_hbm, pidf_hbm, esrc_hbm, efeat_hbm, nid_hbm,
                      lut_hbm, mem_hbm,
                      newts_o, delta_o, featw_o, ssrc_o, sdst_o,
                      bt_v, be_v, mt_v, me_v, bs_v, srcw_v, nsrc_v,
                      tsrc_v, nv_v, ntv_v, newts_v, delta_v, hasf_v,
                      fidx_v, featwT_v, ssrc_v, sdst_v,
                      sem_p, sem_ntv, sem_sdst, sem_w, sem_tsrc,
                      sem_ssrc, sem_f):
    wid = lax.axis_index("s") * _NC + lax.axis_index("c")
    base = wid * _BINS_W
    nv = _BINS_W // _L

    pltpu.sync_copy(nid_hbm.at[pl.ds(base, _BINS_W)], nv_v)
    h_ntv = pltpu.async_copy(lut_hbm.at[nv_v], ntv_v, sem_ntv)
    h_sdst = pltpu.async_copy(mem_hbm.at[nv_v], sdst_v, sem_sdst)

    def fire_body(t, c):
        o = t * _BINS_W
        g = t * _N_PAD + base
        pltpu.async_copy(ptsf_hbm.at[pl.ds(g, _BINS_W)],
                         mt_v.at[pl.ds(o, _BINS_W)], sem_p)
        pltpu.async_copy(pidf_hbm.at[pl.ds(g, _BINS_W)],
                         me_v.at[pl.ds(o, _BINS_W)], sem_p)
        return c
    lax.fori_loop(0, _NW, fire_body, 0)

    def drain_body(t, c):
        o = t * _BINS_W
        g = t * _N_PAD + base
        pltpu.make_async_copy(ptsf_hbm.at[pl.ds(g, _BINS_W)],
                              mt_v.at[pl.ds(o, _BINS_W)], sem_p).wait()
        pltpu.make_async_copy(pidf_hbm.at[pl.ds(g, _BINS_W)],
                              me_v.at[pl.ds(o, _BINS_W)], sem_p).wait()
        return c
    lax.fori_loop(0, _NW, drain_body, 0)

    def vb_init(v, c):
        o = v * _L
        bt_v[pl.ds(o, _L)] = mt_v[pl.ds(o, _L)]
        be_v[pl.ds(o, _L)] = me_v[pl.ds(o, _L)]
        return c
    lax.fori_loop(0, nv, vb_init, 0)

    def merge_body(t, c):
        def vb(v, cc):
            o = v * _L
            bt = bt_v[pl.ds(o, _L)]
            be = be_v[pl.ds(o, _L)]
            mt = mt_v[pl.ds(t * _BINS_W + o, _L)]
            me = me_v[pl.ds(t * _BINS_W + o, _L)]
            w = (mt > bt) | ((mt == bt) & (me > be))
            bt_v[pl.ds(o, _L)] = jnp.where(w, mt, bt)
            be_v[pl.ds(o, _L)] = jnp.where(w, me, be)
            return cc
        lax.fori_loop(0, nv, vb, 0)
        return c
    lax.fori_loop(1, _NW, merge_body, 0)

    def bs_body(v, c):
        o = v * _L
        bs_vec = jnp.maximum(be_v[pl.ds(o, _L)], 0)
        bs_v[pl.ds(o, _L)] = bs_vec
        for j in range(_E_FEAT):
            fidx_v[pl.ds(j * _BINS_W + o, _L)] = bs_vec + j * _N_EDGES
        return c
    lax.fori_loop(0, nv, bs_body, 0)

    for j in range(_E_FEAT):
        pltpu.async_copy(efeat_hbm.at[fidx_v.at[pl.ds(j * _BINS_W, _BINS_W)]],
                         featwT_v.at[pl.ds(j * _BINS_W, _BINS_W)], sem_f)
    pltpu.async_copy(esrc_hbm.at[bs_v], srcw_v, sem_w).wait()
    pltpu.async_copy(nid_hbm.at[srcw_v], nsrc_v, sem_w).wait()
    h_tsrc = pltpu.async_copy(lut_hbm.at[nsrc_v], tsrc_v, sem_tsrc)
    h_ssrc = pltpu.async_copy(mem_hbm.at[nsrc_v], ssrc_v, sem_ssrc)
    h_tsrc.wait()
    h_ntv.wait()

    neg = jnp.full((_L,), -jnp.inf, _f32)
    onev = jnp.full((_L,), 1.0, _f32)
    zerov = jnp.full((_L,), 0.0, _f32)

    def d_body(v, c):
        o = v * _L
        bt = bt_v[pl.ds(o, _L)]
        empty = bt == neg
        delta_v[pl.ds(o, _L)] = bt - tsrc_v[pl.ds(o, _L)]
        hasf_v[pl.ds(o, _L)] = jnp.where(empty, zerov, onev)
        newts_v[pl.ds(o, _L)] = jnp.where(empty, ntv_v[pl.ds(o, _L)], bt)
        return c
    lax.fori_loop(0, nv, d_body, 0)

    h_ssrc.wait()
    h_sdst.wait()
    for j in range(_E_FEAT):
        pltpu.make_async_copy(
            efeat_hbm.at[fidx_v.at[pl.ds(j * _BINS_W, _BINS_W)]],
            featwT_v.at[pl.ds(j * _BINS_W, _BINS_W)], sem_f).wait()

    pltpu.sync_copy(newts_v, newts_o.at[pl.ds(base, _BINS_W)])
    pltpu.sync_copy(delta_v, delta_o.at[pl.ds(base, _BINS_W)])
    pltpu.sync_copy(hasf_v, delta_o.at[pl.ds(_N_PAD + base, _BINS_W)])
    for j in range(_E_FEAT):
        pltpu.sync_copy(featwT_v.at[pl.ds(j * _BINS_W, _BINS_W)],
                        featw_o.at[pl.ds(j * _N_PAD + base, _BINS_W)])
    pltpu.sync_copy(ssrc_v, ssrc_o.at[pl.ds(base, _BINS_W)])
    pltpu.sync_copy(sdst_v, sdst_o.at[pl.ds(base, _BINS_W)])


def _gru_block(ssrc_ref, sdst_ref, featw_ref, dh_ref,
               tw_ref, tb_ref, wis_ref, wid_ref,
               wif_ref, wit_ref, whh_ref, bih_ref, bhh_ref,
               snew_ref):
    f32 = jnp.float32
    cdims = (((0,), (0,)), ((), ()))
    dT = jnp.transpose(dh_ref[...])
    outer = dT[:, :1] * tw_ref[...]
    has_col = dT[:, 1:]
    te = jnp.cos(outer + tb_ref[...])
    sdst = sdst_ref[...]
    gi = (jnp.dot(ssrc_ref[...], wis_ref[...], preferred_element_type=f32)
          + jnp.dot(sdst, wid_ref[...], preferred_element_type=f32)
          + lax.dot_general(featw_ref[...], wif_ref[...], cdims,
                            preferred_element_type=f32)
          + jnp.dot(te, wit_ref[...], preferred_element_type=f32)
          + bih_ref[...])
    gh = jnp.dot(sdst, whh_ref[...], preferred_element_type=f32) + bhh_ref[...]
    D = _MEM_DIM
    r = jax.nn.sigmoid(gi[:, :D] + gh[:, :D])
    z = jax.nn.sigmoid(gi[:, D:2 * D] + gh[:, D:2 * D])
    n = jnp.tanh(gi[:, 2 * D:] + r * gh[:, 2 * D:])
    s_all = (1.0 - z) * n + z * sdst
    has = has_col != 0.0
    snew_ref[...] = jnp.where(has, s_all, sdst)


def kernel(memory, last_update_t, edge_ts, edge_feats, time_w, time_b,
           W_ih, W_hh, b_ih, b_hh, nid, edge_src, edge_dst):
    nid = nid.astype(_i32)
    edge_src = edge_src.astype(_i32)
    edge_dst = edge_dst.astype(_i32)
    nid_pad = jnp.pad(nid, (0, _N_PAD - _N_NODES))

    mesh = plsc.VectorSubcoreMesh(core_axis_name="c", subcore_axis_name="s")
    sc_params = pltpu.CompilerParams(needs_layout_passes=False)

    stage_a = functools.partial(
        pl.kernel,
        out_type=(jax.ShapeDtypeStruct((_NW * _N_PAD,), _f32),
                  jax.ShapeDtypeStruct((_NW * _N_PAD,), _i32)),
        mesh=mesh,
        scratch_types=[pltpu.VMEM((_EDGES_W,), _f32),
                       pltpu.VMEM((_EDGES_W,), _i32),
                       pltpu.VMEM((_N_PAD,), _f32),
                       pltpu.VMEM((_N_PAD,), _i32)],
        compiler_params=sc_params,
    )(_seg_argmax_partial)
    pts, pid = stage_a(edge_ts, edge_dst)

    stage_bc = functools.partial(
        pl.kernel,
        out_type=(jax.ShapeDtypeStruct((_N_PAD,), _f32),
                  jax.ShapeDtypeStruct((2 * _N_PAD,), _f32),
                  jax.ShapeDtypeStruct((_E_FEAT * _N_PAD,), _f32),
                  jax.ShapeDtypeStruct((_N_PAD, _MEM_DIM), _f32),
                  jax.ShapeDtypeStruct((_N_PAD, _MEM_DIM), _f32)),
        mesh=mesh,
        scratch_types=[pltpu.VMEM((_BINS_W,), _f32),
                       pltpu.VMEM((_BINS_W,), _i32),
                       pltpu.VMEM((_NW * _BINS_W,), _f32),
                       pltpu.VMEM((_NW * _BINS_W,), _i32),
                       pltpu.VMEM((_BINS_W,), _i32),
                       pltpu.VMEM((_BINS_W,), _i32),
                       pltpu.VMEM((_BINS_W,), _i32),
                       pltpu.VMEM((_BINS_W,), _f32),
                       pltpu.VMEM((_BINS_W,), _i32),
                       pltpu.VMEM((_BINS_W,), _f32),
                       pltpu.VMEM((_BINS_W,), _f32),
                       pltpu.VMEM((_BINS_W,), _f32),
                       pltpu.VMEM((_BINS_W,), _f32),
                       pltpu.VMEM((_E_FEAT * _BINS_W,), _i32),
                       pltpu.VMEM((_E_FEAT * _BINS_W,), _f32),
                       pltpu.VMEM((_BINS_W, _MEM_DIM), _f32),
                       pltpu.VMEM((_BINS_W, _MEM_DIM), _f32),
                       pltpu.SemaphoreType.DMA,
                       pltpu.SemaphoreType.DMA,
                       pltpu.SemaphoreType.DMA,
                       pltpu.SemaphoreType.DMA,
                       pltpu.SemaphoreType.DMA,
                       pltpu.SemaphoreType.DMA,
                       pltpu.SemaphoreType.DMA],
        compiler_params=pltpu.CompilerParams(needs_layout_passes=False,
                                             use_tc_tiling_on_sc=False),
    )(_merge_and_gather)
    newts, dh_fl, featw_fl, ssrc, sdst = stage_bc(
        pts, pid, edge_src, edge_feats.T.reshape(-1),
        nid_pad, last_update_t, memory)
    featw = featw_fl.reshape(_E_FEAT, _N_PAD)
    dh = dh_fl.reshape(2, _N_PAD)

    B = 512
    grid = (_N_PAD // B,)
    D = _MEM_DIM
    wis = W_ih[:, :D].T
    wid_w = W_ih[:, D:2 * D].T
    wif = W_ih[:, 2 * D:2 * D + _E_FEAT].T
    wit = W_ih[:, 2 * D + _E_FEAT:].T
    whh = W_hh.T
    tw = time_w.reshape(1, _T_DIM)
    tb = time_b.reshape(1, _T_DIM)
    bih = b_ih.reshape(1, 3 * D)
    bhh = b_hh.reshape(1, 3 * D)

    row_spec = lambda w: pl.BlockSpec((B, w), lambda i: (i, 0))
    full_spec = lambda s: pl.BlockSpec(s, lambda i: (0, 0))

    snew = pl.pallas_call(
        _gru_block,
        grid=grid,
        in_specs=[row_spec(D), row_spec(D),
                  pl.BlockSpec((_E_FEAT, B), lambda i: (0, i)),
                  pl.BlockSpec((2, B), lambda i: (0, i)),
                  full_spec((1, _T_DIM)), full_spec((1, _T_DIM)),
                  full_spec((D, 3 * D)), full_spec((D, 3 * D)),
                  full_spec((_E_FEAT, 3 * D)), full_spec((_T_DIM, 3 * D)),
                  full_spec((D, 3 * D)), full_spec((1, 3 * D)),
                  full_spec((1, 3 * D))],
        out_specs=pl.BlockSpec((B, D), lambda i: (i, 0)),
        out_shape=jax.ShapeDtypeStruct((_N_NODES, D), _f32),
    )(ssrc, sdst, featw, dh,
      tw, tb, wis, wid_w, wif, wit, whh, bih, bhh)

    return snew, newts[:_N_NODES]
